# baseline (speedup 1.0000x reference)
import functools

import jax
import jax.numpy as jnp
from jax import lax
from jax.experimental import pallas as pl
from jax.experimental.pallas import tpu as pltpu
from jax.experimental.pallas import tpu_sc as plsc

B = 16384
CTX = 20
NSAMP = 21
SP = 22
DIM = 64
OUTP = 32

NC = 2
NS = 16
NW = NC * NS
BPW = B // NW
NB = 4
NBLK = BPW // NB
R1 = NB * CTX
R2 = NB * SP
LANES = 16
DCH = DIM // LANES


def _cbow_body(ctx_hbm, smp_hbm, w1_hbm, w2_hbm, out_hbm,
               idx1, idx2, r1a, r1b, r2a, r2b, out_v, sem_a, sem_b):
  wid = lax.axis_index("s") * NC + lax.axis_index("c")
  lane = lax.iota(jnp.int32, LANES)
  masks = [lane == jnp.int32(s) for s in range(LANES)]
  rows1 = (r1a, r1b)
  rows2 = (r2a, r2b)
  sems = (sem_a, sem_b)

  pltpu.sync_copy(ctx_hbm.at[pl.ds(wid * (BPW * CTX), BPW * CTX)], idx1)
  pltpu.sync_copy(smp_hbm.at[pl.ds(wid * (BPW * SP), BPW * SP)], idx2)

  def start(blk, slot):
    off1 = pl.multiple_of(blk * R1, 8)
    off2 = pl.multiple_of(blk * R2, 8)
    pltpu.async_copy(w1_hbm.at[idx1.at[pl.ds(off1, R1)]], rows1[slot],
                     sems[slot])
    pltpu.async_copy(w2_hbm.at[idx2.at[pl.ds(off2, R2)]], rows2[slot],
                     sems[slot])

  def wait(slot):
    pltpu.make_async_copy(w1_hbm.at[pl.ds(0, R1)], rows1[slot],
                          sems[slot]).wait()
    pltpu.make_async_copy(w2_hbm.at[pl.ds(0, R2)], rows2[slot],
                          sems[slot]).wait()

  def compute(blk, slot):
    r1 = rows1[slot]
    r2 = rows2[slot]

    def elem(e, _):
      row0 = e * CTX
      h = []
      for d in range(DCH):
        acc = r1[row0, pl.ds(d * LANES, LANES)]
        for r in range(1, CTX):
          acc = acc + r1[row0 + r, pl.ds(d * LANES, LANES)]
        h.append(acc * jnp.float32(1.0 / CTX))
      srow0 = e * SP
      orow = blk * NB + e
      pv = [jnp.zeros((LANES,), jnp.float32) for _ in range(2)]
      for s in range(NSAMP):
        acc = h[0] * r2[srow0 + s, pl.ds(0, LANES)]
        for d in range(1, DCH):
          acc = acc + h[d] * r2[srow0 + s, pl.ds(d * LANES, LANES)]
        g, l = divmod(s, LANES)
        pv[g] = jnp.where(masks[l], lax.broadcast(jnp.sum(acc), (LANES,)),
                          pv[g])
      out_v[orow, pl.ds(0, LANES)] = pv[0]
      out_v[orow, pl.ds(LANES, LANES)] = pv[1]
      return 0

    lax.fori_loop(0, NB, elem, 0)

  start(0, 0)
  def step(i, _):
    for b in range(2):
      blk = i * 2 + b
      wait(b)
      nxt = blk + 1

      @pl.when(nxt < NBLK)
      def _():
        start(nxt, 1 - b)

      compute(blk, b)
    return 0

  lax.fori_loop(0, NBLK // 2, step, 0)
  pltpu.sync_copy(out_v, out_hbm.at[pl.ds(wid * BPW, BPW), :])


@jax.jit
def kernel(context, target, negative_samples, W1, W2):
  ctx_flat = context.astype(jnp.int32).reshape(-1)
  samples = jnp.concatenate(
      [target, negative_samples,
       jnp.zeros((B, SP - NSAMP), target.dtype)], axis=1)
  smp_flat = samples.astype(jnp.int32).reshape(-1)

  mesh = plsc.VectorSubcoreMesh(core_axis_name="c", subcore_axis_name="s")
  k = pl.kernel(
      _cbow_body,
      out_type=jax.ShapeDtypeStruct((B, OUTP), jnp.float32),
      mesh=mesh,
      compiler_params=pltpu.CompilerParams(
          needs_layout_passes=False, use_tc_tiling_on_sc=False),
      scratch_types=[
          pltpu.VMEM((BPW * CTX,), jnp.int32),
          pltpu.VMEM((BPW * SP,), jnp.int32),
          pltpu.VMEM((R1, DIM), jnp.float32),
          pltpu.VMEM((R1, DIM), jnp.float32),
          pltpu.VMEM((R2, DIM), jnp.float32),
          pltpu.VMEM((R2, DIM), jnp.float32),
          pltpu.VMEM((BPW, OUTP), jnp.float32),
          pltpu.SemaphoreType.DMA,
          pltpu.SemaphoreType.DMA,
      ],
  )
  out = k(ctx_flat, smp_flat, W1, W2)
  return out[:, :NSAMP]
